# baseline (speedup 1.0000x reference)
import functools
import jax
import jax.numpy as jnp
from jax import lax
from jax.experimental import pallas as pl
from jax.experimental.pallas import tpu as pltpu
from jax.experimental.pallas import tpu_sc as plsc

_NC = 2
_NS = 16
_NW = _NC * _NS
_B = 16384
_D = 64
_NROWS = 1000000
_RB_FULL = 7812
_RB_PER_W = _RB_FULL // _NW
_RB_EXTRA = _RB_FULL - _RB_PER_W * _NW
_GC = 512
_TAIL_LO = _RB_FULL * 128
_SG = 128


def _iota16():
    return lax.iota(jnp.int32, 16)


def _popcount(m):
    return plsc.all_reduce_population_count(m)[0]


def _process_table(idx_hbm, tt, out_hbm, lo_rb, n_rb, lo, hi,
                   idx_v, l1_v, buf_v, buf2_v, stage_v, stage2_v,
                   vtmp_v, posb_v, posb2_v, semw, semA, semB):
    def win_src(c):
        base_rb = lo_rb + jnp.minimum(c * 4, n_rb - 4)
        c0 = pl.multiple_of(base_rb * 128, 128)
        return tt.at[:, pl.ds(c0, _GC)], (base_rb - lo_rb) * 128

    def start_dma(c, buf):
        src, _ = win_src(c)
        pltpu.async_copy(src, buf, semw)

    start_dma(jnp.int32(0), buf_v)

    def idx_chunk(ci, off):
        pltpu.sync_copy(idx_hbm.at[pl.ds(ci * 4096, 4096)], idx_v)

        def scan_body(i, off):
            for u in range(4):
                b = (i * 4 + u) * 16
                v = idx_v[pl.ds(b, 16)]
                m = (v >= lo) & (v < hi)
                packed = (v - lo) * _B + (_iota16() + ci * 4096 + b)
                plsc.store_compressed(l1_v.at[pl.ds(off, 16)], packed,
                                      mask=m)
                off = off + _popcount(m)
            return off

        return lax.fori_loop(0, 4096 // 64, scan_body, off)

    n = lax.fori_loop(0, 4, idx_chunk, jnp.int32(0))
    nv = (n + 15) // 16

    stages = ((stage_v, posb_v, semA), (stage2_v, posb2_v, semB))

    def drain(g):
        stg = stages[g]
        pltpu.make_async_copy(out_hbm.at[pl.ds(0, _SG)], stg[0],
                              stg[2]).wait()

    def issue_scatter(g):
        stg = stages[g]
        pltpu.async_copy(stg[0], out_hbm.at[stg[1]], stg[2])

    def hit_scan(s, chunk_id, origin_col, buf):
        def vreg_body(j, s):
            hv = l1_v[pl.ds(j * 16, 16)]
            valid = (_iota16() + j * 16) < n
            col = lax.shift_right_logical(hv, 14)
            m = valid & (lax.shift_right_logical(col, 9) == chunk_id)
            p = _popcount(m)

            def work(s):
                plsc.store_compressed(vtmp_v.at[pl.ds(0, 16)], hv, mask=m)

                def hit_body(t, s):
                    k = vtmp_v[pl.ds(t, 16)][0]
                    hcol = lax.shift_right_logical(k, 14)
                    pos = k & (_B - 1)
                    cl = hcol - origin_col
                    slot = s & (_SG - 1)
                    clv = jnp.full((16,), cl, jnp.int32)
                    vals = []
                    for q in range(4):
                        dv = _iota16() + q * 16
                        vals.append(plsc.load_gather(buf, [dv, clv]))
                    par = lax.shift_right_logical(s, 7) & 1

                    def store_to(g):
                        def go(_):
                            stg = stages[g]
                            for q in range(4):
                                stg[0][slot, pl.ds(q * 16, 16)] = vals[q]
                            plsc.store_scatter(
                                stg[1], [jnp.full((16,), slot, jnp.int32)],
                                jnp.full((16,), pos, jnp.int32),
                                mask=_iota16() == 0)
                            return 0
                        return go

                    lax.cond(par == 0, store_to(0), store_to(1), 0)
                    s = s + 1

                    def flush(s):
                        f = lax.shift_right_logical(s, 7)
                        fpar = (f - 1) & 1
                        lax.cond(fpar == 0, lambda _: (issue_scatter(0), 0)[1],
                                 lambda _: (issue_scatter(1), 0)[1], 0)

                        def dr(_):
                            lax.cond(fpar == 0, lambda _: (drain(1), 0)[1],
                                     lambda _: (drain(0), 0)[1], 0)
                            return 0

                        lax.cond(f >= 2, dr, lambda _: 0, 0)
                        return s

                    return lax.cond((s & (_SG - 1)) == 0, flush,
                                    lambda s: s, s)

                return lax.fori_loop(0, p, hit_body, s)

            return lax.cond(p > 0, work, lambda s: s, s)

        return lax.fori_loop(0, nv, vreg_body, s)

    nchunks = (n_rb + 3) // 4
    bufs = (buf_v, buf2_v)

    def pair_body(c2, s):
        for b in range(2):
            c = c2 * 2 + b

            def do(s):
                pltpu.make_async_copy(tt.at[:, pl.ds(0, _GC)],
                                      bufs[b], semw).wait()

                def prefetch(_):
                    start_dma(c + 1, bufs[1 - b])
                    return 0

                lax.cond(c + 1 < nchunks, prefetch, lambda _: 0, 0)
                _, origin = win_src(c)
                return hit_scan(s, c, origin, bufs[b])

            s = lax.cond(c < nchunks, do, lambda s: s, s)
        return s

    s = lax.fori_loop(0, (nchunks + 1) // 2, pair_body, jnp.int32(0))

    def drain_last(s):
        lpar = (lax.shift_right_logical(s, 7) - 1) & 1
        lax.cond(lpar == 0, lambda _: (drain(0), 0)[1],
                 lambda _: (drain(1), 0)[1], 0)
        return s

    s = lax.cond(s >= _SG, drain_last, lambda s: s, s)

    def final_flush(s):
        fpar = lax.shift_right_logical(s, 7) & 1

        def fin(g):
            def go(_):
                stg = stages[g]

                def pad_body(j, _):
                    for q in range(8):
                        stg[0][j, pl.ds(q * 16, 16)] = \
                            stg[0][0, pl.ds(q * 16, 16)]
                    plsc.store_scatter(
                        stg[1], [jnp.full((16,), j, jnp.int32)],
                        stg[1][pl.ds(0, 16)], mask=_iota16() == 0)
                    return 0

                lax.fori_loop(s & (_SG - 1), _SG, pad_body, 0)
                issue_scatter(g)
                drain(g)
                return 0
            return go

        lax.cond(fpar == 0, fin(0), fin(1), 0)
        return s

    lax.cond((s & (_SG - 1)) > 0, final_flush, lambda s: s, s)


def _gather_body(u_idx, i_idx, tu, ti, u_out, i_out,
                 idx_v, l1_v, buf_v, buf2_v, stage_v, stage2_v,
                 vtmp_v, posb_v, posb2_v, semw, semA, semB):
    wid = lax.axis_index("s") * _NC + lax.axis_index("c")
    lo_rb = wid * _RB_PER_W + jnp.minimum(wid, _RB_EXTRA)
    n_rb = _RB_PER_W + (wid < _RB_EXTRA).astype(jnp.int32)
    lo = lo_rb * 128
    hi = lo + n_rb * 128

    def zrow(j, _):
        z = jnp.zeros((16,), jnp.float32)
        for q in range(4, 8):
            stage_v[j, pl.ds(q * 16, 16)] = z
            stage2_v[j, pl.ds(q * 16, 16)] = z
        return 0

    lax.fori_loop(0, _SG, zrow, 0)

    _process_table(u_idx, tu, u_out, lo_rb, n_rb, lo, hi,
                   idx_v, l1_v, buf_v, buf2_v, stage_v, stage2_v,
                   vtmp_v, posb_v, posb2_v, semw, semA, semB)
    _process_table(i_idx, ti, i_out, lo_rb, n_rb, lo, hi,
                   idx_v, l1_v, buf_v, buf2_v, stage_v, stage2_v,
                   vtmp_v, posb_v, posb2_v, semw, semA, semB)


@functools.lru_cache(maxsize=None)
def _sc_gather():
    return pl.kernel(
        _gather_body,
        out_type=(
            jax.ShapeDtypeStruct((_B, 128), jnp.float32),
            jax.ShapeDtypeStruct((_B, 128), jnp.float32),
        ),
        mesh=plsc.VectorSubcoreMesh(
            core_axis_name="c", subcore_axis_name="s",
            num_cores=_NC, num_subcores=_NS),
        scratch_types=[
            pltpu.VMEM((4096,), jnp.int32),
            pltpu.VMEM((_B + 16,), jnp.int32),
            pltpu.VMEM((_D, _GC), jnp.float32),
            pltpu.VMEM((_D, _GC), jnp.float32),
            pltpu.VMEM((_SG, 128), jnp.float32),
            pltpu.VMEM((_SG, 128), jnp.float32),
            pltpu.VMEM((32,), jnp.int32),
            pltpu.VMEM((_SG,), jnp.int32),
            pltpu.VMEM((_SG,), jnp.int32),
            pltpu.SemaphoreType.DMA,
            pltpu.SemaphoreType.DMA,
            pltpu.SemaphoreType.DMA,
        ],
        compiler_params=pltpu.CompilerParams(needs_layout_passes=False),
    )


def _fix_tail(emb, idx2, mini):
    tail_off = idx2 - _TAIL_LO
    oh = (tail_off == lax.broadcasted_iota(jnp.int32, (1, _D), 1))
    fix = oh.astype(jnp.float32) @ mini
    return jnp.where(idx2 >= _TAIL_LO, fix, emb)


def _mlp_body(ue, ie, uidx, iidx, umini, imini,
              w1u, w1i, b1, w2, b2, w3, b3, wo, bo, out):
    uef = _fix_tail(ue[...], uidx[...], umini[...])
    ief = _fix_tail(ie[...], iidx[...], imini[...])
    h = jnp.maximum(uef @ w1u[...] + ief @ w1i[...] + b1[...], 0.0)
    h = jnp.maximum(h @ w2[...] + b2[...], 0.0)
    h = jnp.maximum(h @ w3[...] + b3[...], 0.0)
    o = jnp.sum(h * wo[...], axis=1, keepdims=True) + bo[...]
    out[...] = jax.nn.sigmoid(o)


def _mlp(ue, ie, uidx, iidx, umini, imini,
         w1u, w1i, b1, w2, b2, w3, b3, wo, bo, blk=2048):
    grid = _B // blk
    full = lambda shape: pl.BlockSpec(shape, lambda i: (0, 0))
    return pl.pallas_call(
        _mlp_body,
        grid=(grid,),
        in_specs=[
            pl.BlockSpec((blk, 128), lambda i: (i, 0)),
            pl.BlockSpec((blk, 128), lambda i: (i, 0)),
            pl.BlockSpec((blk, 1), lambda i: (i, 0)),
            pl.BlockSpec((blk, 1), lambda i: (i, 0)),
            full((_D, 128)), full((_D, 128)),
            full((128, 128)), full((128, 128)), full((1, 128)),
            full((128, 64)), full((1, 64)),
            full((64, 32)), full((1, 32)),
            full((1, 32)), full((1, 1)),
        ],
        out_specs=pl.BlockSpec((blk, 1), lambda i: (i, 0)),
        out_shape=jax.ShapeDtypeStruct((_B, 1), jnp.float32),
    )(ue, ie, uidx, iidx, umini, imini,
      w1u, w1i, b1, w2, b2, w3, b3, wo, bo)


@jax.jit
def kernel(user_input, item_input, user_table, item_table,
           W1, b1, W2, b2, W3, b3, Wo, bo):
    ue, ie = _sc_gather()(user_input, item_input,
                          user_table.T, item_table.T)
    zpad = jnp.zeros((_D, 128), jnp.float32)
    w1u = jnp.concatenate([W1[:_D], zpad], axis=0)
    w1i = jnp.concatenate([W1[_D:], zpad], axis=0)
    pad64 = ((0, 0), (0, 64))
    umini = jnp.pad(user_table[_TAIL_LO:], pad64)
    imini = jnp.pad(item_table[_TAIL_LO:], pad64)
    return _mlp(
        ue, ie,
        user_input.reshape(_B, 1), item_input.reshape(_B, 1),
        umini, imini,
        w1u, w1i, b1.reshape(1, 128),
        W2, b2.reshape(1, 64),
        W3, b3.reshape(1, 32),
        Wo.reshape(1, 32), bo.reshape(1, 1),
    )


# speedup vs baseline: 1.0047x; 1.0047x over previous
import functools
import jax
import jax.numpy as jnp
from jax import lax
from jax.experimental import pallas as pl
from jax.experimental.pallas import tpu as pltpu
from jax.experimental.pallas import tpu_sc as plsc

_NC = 2
_NS = 16
_NW = _NC * _NS
_B = 16384
_D = 64
_NROWS = 1000000
_RB_FULL = 7812
_RB_PER_W = _RB_FULL // _NW
_RB_EXTRA = _RB_FULL - _RB_PER_W * _NW
_GC = 512
_TAIL_LO = _RB_FULL * 128
_SG = 128


def _iota16():
    return lax.iota(jnp.int32, 16)


def _popcount(m):
    return plsc.all_reduce_population_count(m)[0]


def _process_table(idx_hbm, tt, out_hbm, lo_rb, n_rb, lo, hi,
                   idx_v, l1_v, buf_v, buf2_v, stage_v, stage2_v,
                   vtmp_v, posb_v, posb2_v, semw, semA, semB):
    def win_src(c):
        base_rb = lo_rb + jnp.minimum(c * 4, n_rb - 4)
        c0 = pl.multiple_of(base_rb * 128, 128)
        return tt.at[:, pl.ds(c0, _GC)], (base_rb - lo_rb) * 128

    def start_dma(c, buf):
        src, _ = win_src(c)
        pltpu.async_copy(src, buf, semw)

    start_dma(jnp.int32(0), buf_v)

    def idx_chunk(ci, off):
        pltpu.sync_copy(idx_hbm.at[pl.ds(ci * 4096, 4096)], idx_v)

        def scan_body(i, off):
            for u in range(4):
                b = (i * 4 + u) * 16
                v = idx_v[pl.ds(b, 16)]
                m = (v >= lo) & (v < hi)
                packed = (v - lo) * _B + (_iota16() + ci * 4096 + b)
                plsc.store_compressed(l1_v.at[pl.ds(off, 16)], packed,
                                      mask=m)
                off = off + _popcount(m)
            return off

        return lax.fori_loop(0, 4096 // 64, scan_body, off)

    n = lax.fori_loop(0, 4, idx_chunk, jnp.int32(0))
    nv = (n + 15) // 16

    stages = ((stage_v, posb_v, semA), (stage2_v, posb2_v, semB))

    def drain(g):
        stg = stages[g]
        pltpu.make_async_copy(out_hbm.at[pl.ds(0, _SG)], stg[0],
                              stg[2]).wait()

    def issue_scatter(g):
        stg = stages[g]
        pltpu.async_copy(stg[0], out_hbm.at[stg[1]], stg[2])

    def hit_scan(s, chunk_id, origin_col, buf):
        def vreg_body(j, s):
            hv = l1_v[pl.ds(j * 16, 16)]
            valid = (_iota16() + j * 16) < n
            col = lax.shift_right_logical(hv, 14)
            m = valid & (lax.shift_right_logical(col, 9) == chunk_id)
            p = _popcount(m)

            def work(s):
                plsc.store_compressed(vtmp_v.at[pl.ds(0, 16)], hv, mask=m)

                def hit_body(t, s):
                    k = vtmp_v[pl.ds(t, 16)][0]
                    hcol = lax.shift_right_logical(k, 14)
                    pos = k & (_B - 1)
                    cl = hcol - origin_col
                    slot = s & (_SG - 1)
                    clv = jnp.full((16,), cl, jnp.int32)
                    vals = []
                    for q in range(4):
                        dv = _iota16() + q * 16
                        vals.append(plsc.load_gather(buf, [dv, clv]))
                    par = lax.shift_right_logical(s, 7) & 1

                    def store_to(g):
                        def go(_):
                            stg = stages[g]
                            for q in range(4):
                                stg[0][slot, pl.ds(q * 16, 16)] = vals[q]
                            plsc.store_scatter(
                                stg[1], [jnp.full((16,), slot, jnp.int32)],
                                jnp.full((16,), pos, jnp.int32),
                                mask=_iota16() == 0)
                            return 0
                        return go

                    lax.cond(par == 0, store_to(0), store_to(1), 0)
                    s = s + 1

                    def flush(s):
                        f = lax.shift_right_logical(s, 7)
                        fpar = (f - 1) & 1
                        lax.cond(fpar == 0, lambda _: (issue_scatter(0), 0)[1],
                                 lambda _: (issue_scatter(1), 0)[1], 0)

                        def dr(_):
                            lax.cond(fpar == 0, lambda _: (drain(1), 0)[1],
                                     lambda _: (drain(0), 0)[1], 0)
                            return 0

                        lax.cond(f >= 2, dr, lambda _: 0, 0)
                        return s

                    return lax.cond((s & (_SG - 1)) == 0, flush,
                                    lambda s: s, s)

                return lax.fori_loop(0, p, hit_body, s)

            return lax.cond(p > 0, work, lambda s: s, s)

        return lax.fori_loop(0, nv, vreg_body, s)

    nchunks = (n_rb + 3) // 4
    bufs = (buf_v, buf2_v)

    def pair_body(c2, s):
        for b in range(2):
            c = c2 * 2 + b

            def do(s):
                pltpu.make_async_copy(tt.at[:, pl.ds(0, _GC)],
                                      bufs[b], semw).wait()

                def prefetch(_):
                    start_dma(c + 1, bufs[1 - b])
                    return 0

                lax.cond(c + 1 < nchunks, prefetch, lambda _: 0, 0)
                _, origin = win_src(c)
                return hit_scan(s, c, origin, bufs[b])

            s = lax.cond(c < nchunks, do, lambda s: s, s)
        return s

    s = lax.fori_loop(0, (nchunks + 1) // 2, pair_body, jnp.int32(0))

    def drain_last(s):
        lpar = (lax.shift_right_logical(s, 7) - 1) & 1
        lax.cond(lpar == 0, lambda _: (drain(0), 0)[1],
                 lambda _: (drain(1), 0)[1], 0)
        return s

    s = lax.cond(s >= _SG, drain_last, lambda s: s, s)

    def final_flush(s):
        fpar = lax.shift_right_logical(s, 7) & 1

        def fin(g):
            def go(_):
                stg = stages[g]

                def pad_body(j, _):
                    for q in range(8):
                        stg[0][j, pl.ds(q * 16, 16)] = \
                            stg[0][0, pl.ds(q * 16, 16)]
                    plsc.store_scatter(
                        stg[1], [jnp.full((16,), j, jnp.int32)],
                        stg[1][pl.ds(0, 16)], mask=_iota16() == 0)
                    return 0

                lax.fori_loop(s & (_SG - 1), _SG, pad_body, 0)
                issue_scatter(g)
                drain(g)
                return 0
            return go

        lax.cond(fpar == 0, fin(0), fin(1), 0)
        return s

    lax.cond((s & (_SG - 1)) > 0, final_flush, lambda s: s, s)


def _gather_body(u_idx, i_idx, tu, ti, u_out, i_out,
                 idx_v, l1_v, buf_v, buf2_v, stage_v, stage2_v,
                 vtmp_v, posb_v, posb2_v, semw, semA, semB):
    wid = lax.axis_index("s") * _NC + lax.axis_index("c")
    lo_rb = wid * _RB_PER_W + jnp.minimum(wid, _RB_EXTRA)
    n_rb = _RB_PER_W + (wid < _RB_EXTRA).astype(jnp.int32)
    lo = lo_rb * 128
    hi = lo + n_rb * 128

    def zrow(j, _):
        z = jnp.zeros((16,), jnp.float32)
        for q in range(4, 8):
            stage_v[j, pl.ds(q * 16, 16)] = z
            stage2_v[j, pl.ds(q * 16, 16)] = z
        return 0

    lax.fori_loop(0, _SG, zrow, 0)

    _process_table(u_idx, tu, u_out, lo_rb, n_rb, lo, hi,
                   idx_v, l1_v, buf_v, buf2_v, stage_v, stage2_v,
                   vtmp_v, posb_v, posb2_v, semw, semA, semB)
    _process_table(i_idx, ti, i_out, lo_rb, n_rb, lo, hi,
                   idx_v, l1_v, buf_v, buf2_v, stage_v, stage2_v,
                   vtmp_v, posb_v, posb2_v, semw, semA, semB)


@functools.lru_cache(maxsize=None)
def _sc_gather():
    return pl.kernel(
        _gather_body,
        out_type=(
            jax.ShapeDtypeStruct((_B, 128), jnp.float32),
            jax.ShapeDtypeStruct((_B, 128), jnp.float32),
        ),
        mesh=plsc.VectorSubcoreMesh(
            core_axis_name="c", subcore_axis_name="s",
            num_cores=_NC, num_subcores=_NS),
        scratch_types=[
            pltpu.VMEM((4096,), jnp.int32),
            pltpu.VMEM((_B + 16,), jnp.int32),
            pltpu.VMEM((_D, _GC), jnp.float32),
            pltpu.VMEM((_D, _GC), jnp.float32),
            pltpu.VMEM((_SG, 128), jnp.float32),
            pltpu.VMEM((_SG, 128), jnp.float32),
            pltpu.VMEM((32,), jnp.int32),
            pltpu.VMEM((_SG,), jnp.int32),
            pltpu.VMEM((_SG,), jnp.int32),
            pltpu.SemaphoreType.DMA,
            pltpu.SemaphoreType.DMA,
            pltpu.SemaphoreType.DMA,
        ],
        compiler_params=pltpu.CompilerParams(needs_layout_passes=False),
    )


def _fix_tail(emb, idx2, mini):
    tail_off = idx2 - _TAIL_LO
    oh = (tail_off == lax.broadcasted_iota(jnp.int32, (1, _D), 1))
    fix = oh.astype(jnp.float32) @ mini
    return jnp.where(idx2 >= _TAIL_LO, fix, emb)


def _mlp_body(ue, ie, uidx, iidx, umini, imini,
              w1u, w1i, b1, w2, b2, w3, b3, wo, bo, out):
    uef = _fix_tail(ue[...], uidx[...], umini[...])
    ief = _fix_tail(ie[...], iidx[...], imini[...])
    h = jnp.maximum(uef @ w1u[...] + ief @ w1i[...] + b1[...], 0.0)
    h = jnp.maximum(h @ w2[...] + b2[...], 0.0)
    h = jnp.maximum(h @ w3[...] + b3[...], 0.0)
    o = jnp.sum(h * wo[...], axis=1, keepdims=True) + bo[...]
    out[...] = jax.nn.sigmoid(o)


def _mlp(ue, ie, uidx, iidx, umini, imini,
         w1u, w1i, b1, w2, b2, w3, b3, wo, bo, blk=4096):
    grid = _B // blk
    full = lambda shape: pl.BlockSpec(shape, lambda i: (0, 0))
    return pl.pallas_call(
        _mlp_body,
        grid=(grid,),
        in_specs=[
            pl.BlockSpec((blk, 128), lambda i: (i, 0)),
            pl.BlockSpec((blk, 128), lambda i: (i, 0)),
            pl.BlockSpec((blk, 1), lambda i: (i, 0)),
            pl.BlockSpec((blk, 1), lambda i: (i, 0)),
            full((_D, 128)), full((_D, 128)),
            full((128, 128)), full((128, 128)), full((1, 128)),
            full((128, 64)), full((1, 64)),
            full((64, 32)), full((1, 32)),
            full((1, 32)), full((1, 1)),
        ],
        out_specs=pl.BlockSpec((blk, 1), lambda i: (i, 0)),
        out_shape=jax.ShapeDtypeStruct((_B, 1), jnp.float32),
    )(ue, ie, uidx, iidx, umini, imini,
      w1u, w1i, b1, w2, b2, w3, b3, wo, bo)


@jax.jit
def kernel(user_input, item_input, user_table, item_table,
           W1, b1, W2, b2, W3, b3, Wo, bo):
    ue, ie = _sc_gather()(user_input, item_input,
                          user_table.T, item_table.T)
    zpad = jnp.zeros((_D, 128), jnp.float32)
    w1u = jnp.concatenate([W1[:_D], zpad], axis=0)
    w1i = jnp.concatenate([W1[_D:], zpad], axis=0)
    pad64 = ((0, 0), (0, 64))
    umini = jnp.pad(user_table[_TAIL_LO:], pad64)
    imini = jnp.pad(item_table[_TAIL_LO:], pad64)
    return _mlp(
        ue, ie,
        user_input.reshape(_B, 1), item_input.reshape(_B, 1),
        umini, imini,
        w1u, w1i, b1.reshape(1, 128),
        W2, b2.reshape(1, 64),
        W3, b3.reshape(1, 32),
        Wo.reshape(1, 32), bo.reshape(1, 1),
    )
